# baseline (speedup 1.0000x reference)
import functools
import jax
import jax.numpy as jnp
from jax import lax
from jax.experimental import pallas as pl
from jax.experimental.pallas import tpu as pltpu
from jax.experimental.pallas import tpu_sc as plsc

_B = 16384
_D = 64
_NC = 2
_NS = 16
_NW = _NC * _NS
_BPW = _B // _NW
_LANES = 16
_VPR = _D // _LANES


def _sc_body(tensor_hbm, targets_hbm, centers_hbm, out_hbm,
             idx_v, rows_v, t_v, acc_v, gsem):
  wid = lax.axis_index("s") * _NC + lax.axis_index("c")
  base = wid * _BPW

  pltpu.sync_copy(targets_hbm.at[pl.ds(base, _BPW)], idx_v)
  gather = pltpu.async_copy(centers_hbm.at[idx_v], rows_v, gsem)
  pltpu.sync_copy(tensor_hbm.at[pl.ds(base, _BPW), :], t_v)
  gather.wait()

  def row_step(r, acc):
    for j in range(_VPR):
      d = t_v[r, pl.ds(j * _LANES, _LANES)] - rows_v[r, pl.ds(j * _LANES, _LANES)]
      acc = acc + d * d
    return acc

  acc = lax.fori_loop(0, _BPW, row_step, jnp.zeros((_LANES,), jnp.float32))
  acc_v[...] = acc
  pltpu.sync_copy(acc_v, out_hbm.at[wid])


@jax.jit
def kernel(tensor, targets, centers):
  targets = targets.astype(jnp.int32)
  partials = pl.kernel(
      _sc_body,
      out_type=jax.ShapeDtypeStruct((_NW, _LANES), jnp.float32),
      mesh=plsc.VectorSubcoreMesh(core_axis_name="c", subcore_axis_name="s"),
      scratch_types=[
          pltpu.VMEM((_BPW,), jnp.int32),
          pltpu.VMEM((_BPW, _D), jnp.float32),
          pltpu.VMEM((_BPW, _D), jnp.float32),
          pltpu.VMEM((_LANES,), jnp.float32),
          pltpu.SemaphoreType.DMA,
      ],
      compiler_params=pltpu.CompilerParams(use_tc_tiling_on_sc=False),
  )(tensor, targets, centers)
  return 0.5 * jnp.sum(partials)
